# baseline (speedup 1.0000x reference)
import jax
import jax.numpy as jnp
from jax import lax
from jax.experimental import pallas as pl
from jax.experimental.pallas import tpu as pltpu
from jax.experimental.pallas import tpu_sc as plsc

_TEMP = 0.3
_EPS = 1e-09
_EPS2 = 1e-10
_N = 4096
_V = 64
_Q = 2
_D = 64
_J = _V * _Q
_NC = 2
_NS = 16
_IPT = _N // _NS


def _sc_meta_kernel(ti_hbm, k1_hbm, k2_hbm, cnt_hbm,
                    ti_v, flags_v, rank_v, utp_v, k1_v, k2_v, cnt_v):
    cid = lax.axis_index("c")
    sid = lax.axis_index("s")
    wid = sid * _NC + cid
    zeros16 = jnp.zeros((16,), jnp.int32)
    ones16 = jnp.ones((16,), jnp.int32)

    @pl.when(wid == 0)
    def _():
        pltpu.sync_copy(ti_hbm, ti_v)
        for c in range(_V // 16):
            flags_v[pl.ds(c * 16, 16)] = zeros16
            utp_v[pl.ds(c * 16, 16)] = zeros16
        for c in range(_N // 16):
            t = ti_v[pl.ds(c * 16, 16)]
            plsc.store_scatter(flags_v, [t], ones16)
        carry = jnp.int32(0)
        for c in range(_V // 16):
            ch = flags_v[pl.ds(c * 16, 16)]
            inc = plsc.cumsum(ch)
            rank_v[pl.ds(c * 16, 16)] = inc - ch + carry
            carry = carry + jnp.sum(ch)
        u_cnt = carry
        for c in range(_V // 16):
            ch = flags_v[pl.ds(c * 16, 16)]
            rk = rank_v[pl.ds(c * 16, 16)]
            vals = lax.iota(jnp.int32, 16) + (c * 16)
            plsc.store_scatter(utp_v, [rk], vals, mask=(ch != 0))
        for c in range(_V // 16):
            r = rank_v[pl.ds(c * 16, 16)]
            u1 = plsc.load_gather(utp_v, [lax.shift_right_logical(r, 1)])
            k1 = u1 * 2 + (r & 1)
            r2 = r + u_cnt
            u2 = plsc.load_gather(utp_v, [lax.shift_right_logical(r2, 1)])
            k2 = u2 * 2 + (r2 & 1)
            k1_v[pl.ds(c * 16, 16)] = k1
            k2_v[pl.ds(c * 16, 16)] = k2
        pltpu.sync_copy(k1_v, k1_hbm)
        pltpu.sync_copy(k2_v, k2_hbm)
        onesf = jnp.ones((16,), jnp.float32)
        for c in range(_J // 16):
            cnt_v[pl.ds(c * 16, 16)] = jnp.zeros((16,), jnp.float32)
        for c in range(_J // 16):
            j = lax.iota(jnp.int32, 16) + c * 16
            s = lax.shift_right_logical(j, 1)
            u = plsc.load_gather(utp_v, [s])
            g = u * 2 + (j & 1)
            validm = j < 2 * u_cnt
            plsc.addupdate_scatter(cnt_v, [g], onesf, mask=validm)
        pltpu.sync_copy(cnt_v, cnt_hbm)


def _tc_dense_kernel(x_ref, y2_ref, ti_ref, k1_ref, k2_ref, cnt_ref, out_ref):
    x = x_ref[...]
    y2 = y2_ref[...]
    ti = ti_ref[...]
    logits = lax.dot_general(x, y2, (((1,), (1,)), ((), ())),
                             preferred_element_type=jnp.float32)
    e = jnp.exp(logits * (1.0 / _TEMP))
    vals = lax.broadcasted_iota(jnp.int32, (1, _V), 1)
    onehot = (ti == vals).astype(jnp.float32)
    s = lax.dot_general(onehot, e, (((0,), (0,)), ((), ())),
                        preferred_element_type=jnp.float32)
    kk = lax.broadcasted_iota(jnp.int32, (1, _J), 1)
    match = (kk == k1_ref[...]) | (kk == k2_ref[...])
    num = jnp.sum(s * match.astype(jnp.float32), axis=(0, 1), keepdims=True)
    colsum = jnp.sum(s, axis=0, keepdims=True)
    total = jnp.sum(colsum * cnt_ref[...], axis=(0, 1), keepdims=True)
    den = total - num
    out_ref[...] = -jnp.log(num / (den + _EPS) + _EPS2)


@jax.jit
def kernel(x, track_idxs, y):
    ti = track_idxs.astype(jnp.int32)
    y2 = y.reshape(_J, _D)
    sc = pl.kernel(
        _sc_meta_kernel,
        out_type=(
            jax.ShapeDtypeStruct((_V,), jnp.int32),
            jax.ShapeDtypeStruct((_V,), jnp.int32),
            jax.ShapeDtypeStruct((_J,), jnp.float32),
        ),
        mesh=plsc.VectorSubcoreMesh(core_axis_name="c", subcore_axis_name="s",
                                    num_cores=_NC, num_subcores=_NS),
        compiler_params=pltpu.CompilerParams(needs_layout_passes=False),
        scratch_types=[
            pltpu.VMEM((_N,), jnp.int32),
            pltpu.VMEM((_V,), jnp.int32),
            pltpu.VMEM((_V,), jnp.int32),
            pltpu.VMEM((_V,), jnp.int32),
            pltpu.VMEM((_V,), jnp.int32),
            pltpu.VMEM((_V,), jnp.int32),
            pltpu.VMEM((_J,), jnp.float32),
        ],
    )
    k1v, k2v, cnt = sc(ti)
    out = pl.pallas_call(
        _tc_dense_kernel,
        out_shape=jax.ShapeDtypeStruct((1, 1), jnp.float32),
    )(x, y2, ti.reshape(_N, 1), k1v.reshape(_V, 1), k2v.reshape(_V, 1),
      cnt.reshape(1, _J))
    return out.reshape(1)


# speedup vs baseline: 1.1104x; 1.1104x over previous
import jax
import jax.numpy as jnp
from jax import lax
from jax.experimental import pallas as pl
from jax.experimental.pallas import tpu as pltpu
from jax.experimental.pallas import tpu_sc as plsc

_TEMP = 0.3
_EPS = 1e-09
_EPS2 = 1e-10
_N = 4096
_V = 64
_Q = 2
_D = 64
_J = _V * _Q
_NC = 2
_NS = 16
_MR = 72


def _sc_meta_kernel(ti_hbm, meta_hbm,
                    ti_v, flags_v, rank_v, utp_v, meta_v):
    cid = lax.axis_index("c")
    sid = lax.axis_index("s")
    wid = sid * _NC + cid
    zeros16 = jnp.zeros((16,), jnp.int32)
    ones16 = jnp.ones((16,), jnp.int32)

    @pl.when(wid == 0)
    def _():
        pltpu.sync_copy(ti_hbm, ti_v)
        zf = jnp.zeros((16,), jnp.float32)
        onesf = jnp.ones((16,), jnp.float32)
        for r in range(_MR):
            for c in range(_J // 16):
                meta_v[r, pl.ds(c * 16, 16)] = zf
        for c in range(_V // 16):
            flags_v[pl.ds(c * 16, 16)] = zeros16
            utp_v[pl.ds(c * 16, 16)] = zeros16
        for c in range(_N // 16):
            t = ti_v[pl.ds(c * 16, 16)]
            plsc.store_scatter(flags_v, [t], ones16)
        carry = jnp.int32(0)
        for c in range(_V // 16):
            ch = flags_v[pl.ds(c * 16, 16)]
            inc = plsc.cumsum(ch)
            rank_v[pl.ds(c * 16, 16)] = inc - ch + carry
            carry = carry + jnp.sum(ch)
        u_cnt = carry
        for c in range(_V // 16):
            ch = flags_v[pl.ds(c * 16, 16)]
            rk = rank_v[pl.ds(c * 16, 16)]
            vals = lax.iota(jnp.int32, 16) + (c * 16)
            plsc.store_scatter(utp_v, [rk], vals, mask=(ch != 0))
        for c in range(_V // 16):
            vrow = lax.iota(jnp.int32, 16) + (c * 16)
            r = rank_v[pl.ds(c * 16, 16)]
            u1 = plsc.load_gather(utp_v, [lax.shift_right_logical(r, 1)])
            k1 = u1 * 2 + (r & 1)
            r2 = r + u_cnt
            u2 = plsc.load_gather(utp_v, [lax.shift_right_logical(r2, 1)])
            k2 = u2 * 2 + (r2 & 1)
            plsc.store_scatter(meta_v, [vrow, k1], onesf)
            plsc.store_scatter(meta_v, [vrow, k2], onesf)
        row64 = jnp.full((16,), _V, dtype=jnp.int32)
        for c in range(_J // 16):
            j = lax.iota(jnp.int32, 16) + c * 16
            s = lax.shift_right_logical(j, 1)
            u = plsc.load_gather(utp_v, [s])
            g = u * 2 + (j & 1)
            validm = j < 2 * u_cnt
            plsc.addupdate_scatter(meta_v, [row64, g], onesf, mask=validm)
        pltpu.sync_copy(meta_v, meta_hbm)


def _tc_dense_kernel(x_ref, y2_ref, ti_ref, s_ref):
    x = x_ref[...]
    y2 = y2_ref[...]
    ti = ti_ref[...]
    logits = lax.dot_general(x, y2, (((1,), (1,)), ((), ())),
                             preferred_element_type=jnp.float32)
    e = jnp.exp(logits * (1.0 / _TEMP))
    vals = lax.broadcasted_iota(jnp.int32, (1, _V), 1)
    onehot = (ti == vals).astype(jnp.float32)
    s_ref[...] = lax.dot_general(onehot, e, (((0,), (0,)), ((), ())),
                                 preferred_element_type=jnp.float32)


def _tc_combine_kernel(s_ref, meta_ref, out_ref):
    s = s_ref[...]
    meta = meta_ref[...]
    match = lax.slice(meta, (0, 0), (_V, _J))
    cnt = lax.slice(meta, (_V, 0), (_V + 1, _J))
    num = jnp.sum(s * match, axis=(0, 1), keepdims=True)
    colsum = jnp.sum(s, axis=0, keepdims=True)
    total = jnp.sum(colsum * cnt, axis=(0, 1), keepdims=True)
    den = total - num
    out_ref[...] = -jnp.log(num / (den + _EPS) + _EPS2)


@jax.jit
def kernel(x, track_idxs, y):
    ti = track_idxs.astype(jnp.int32)
    y2 = y.reshape(_J, _D)
    sc = pl.kernel(
        _sc_meta_kernel,
        out_type=jax.ShapeDtypeStruct((_MR, _J), jnp.float32),
        mesh=plsc.VectorSubcoreMesh(core_axis_name="c", subcore_axis_name="s",
                                    num_cores=_NC, num_subcores=_NS),
        compiler_params=pltpu.CompilerParams(needs_layout_passes=False),
        scratch_types=[
            pltpu.VMEM((_N,), jnp.int32),
            pltpu.VMEM((_V,), jnp.int32),
            pltpu.VMEM((_V,), jnp.int32),
            pltpu.VMEM((_V,), jnp.int32),
            pltpu.VMEM((_MR, _J), jnp.float32),
        ],
    )
    meta = sc(ti)
    s = pl.pallas_call(
        _tc_dense_kernel,
        out_shape=jax.ShapeDtypeStruct((_V, _J), jnp.float32),
    )(x, y2, ti.reshape(_N, 1))
    out = pl.pallas_call(
        _tc_combine_kernel,
        out_shape=jax.ShapeDtypeStruct((1, 1), jnp.float32),
    )(s, meta)
    return out.reshape(1)


# speedup vs baseline: 1.1265x; 1.0146x over previous
import jax
import jax.numpy as jnp
from jax import lax
from jax.experimental import pallas as pl
from jax.experimental.pallas import tpu as pltpu
from jax.experimental.pallas import tpu_sc as plsc

_TEMP = 0.3
_EPS = 1e-09
_EPS2 = 1e-10
_N = 4096
_V = 64
_Q = 2
_D = 64
_J = _V * _Q
_NC = 2
_NS = 16
_MR = 72


def _sc_meta_kernel(ti_hbm, meta_hbm,
                    ti_v, flags_v, rank_v, utp_v, meta_v):
    cid = lax.axis_index("c")
    sid = lax.axis_index("s")
    wid = sid * _NC + cid
    zeros16 = jnp.zeros((16,), jnp.int32)
    ones16 = jnp.ones((16,), jnp.int32)

    @pl.when(wid == 0)
    def _():
        pltpu.sync_copy(ti_hbm, ti_v)
        zf = jnp.zeros((16,), jnp.float32)
        onesf = jnp.ones((16,), jnp.float32)
        for r in range(_MR):
            for c in range(_J // 16):
                meta_v[r, pl.ds(c * 16, 16)] = zf
        for c in range(_V // 16):
            flags_v[pl.ds(c * 16, 16)] = zeros16
            utp_v[pl.ds(c * 16, 16)] = zeros16
        for c in range(_N // 16):
            t = ti_v[pl.ds(c * 16, 16)]
            plsc.store_scatter(flags_v, [t], ones16)
        carry = jnp.int32(0)
        for c in range(_V // 16):
            ch = flags_v[pl.ds(c * 16, 16)]
            inc = plsc.cumsum(ch)
            rank_v[pl.ds(c * 16, 16)] = inc - ch + carry
            carry = carry + jnp.sum(ch)
        u_cnt = carry
        for c in range(_V // 16):
            ch = flags_v[pl.ds(c * 16, 16)]
            rk = rank_v[pl.ds(c * 16, 16)]
            vals = lax.iota(jnp.int32, 16) + (c * 16)
            plsc.store_scatter(utp_v, [rk], vals, mask=(ch != 0))
        for c in range(_V // 16):
            vrow = lax.iota(jnp.int32, 16) + (c * 16)
            r = rank_v[pl.ds(c * 16, 16)]
            u1 = plsc.load_gather(utp_v, [lax.shift_right_logical(r, 1)])
            k1 = u1 * 2 + (r & 1)
            r2 = r + u_cnt
            u2 = plsc.load_gather(utp_v, [lax.shift_right_logical(r2, 1)])
            k2 = u2 * 2 + (r2 & 1)
            plsc.store_scatter(meta_v, [vrow, k1], onesf)
            plsc.store_scatter(meta_v, [vrow, k2], onesf)
        row64 = jnp.full((16,), _V, dtype=jnp.int32)
        for c in range(_J // 16):
            j = lax.iota(jnp.int32, 16) + c * 16
            s = lax.shift_right_logical(j, 1)
            u = plsc.load_gather(utp_v, [s])
            g = u * 2 + (j & 1)
            validm = j < 2 * u_cnt
            plsc.addupdate_scatter(meta_v, [row64, g], onesf, mask=validm)
        pltpu.sync_copy(meta_v, meta_hbm)


def _tc_dense_kernel(x_ref, y_ref, ti_ref, s_ref):
    x = x_ref[...]
    y2 = y_ref[...].reshape(_J, _D)
    ti = ti_ref[...].astype(jnp.int32)
    logits = lax.dot_general(x, y2, (((1,), (1,)), ((), ())),
                             preferred_element_type=jnp.float32)
    e = jnp.exp(logits * (1.0 / _TEMP))
    vals = lax.broadcasted_iota(jnp.int32, (1, _V), 1)
    onehot = (ti == vals).astype(jnp.float32)
    s_ref[...] = lax.dot_general(onehot, e, (((0,), (0,)), ((), ())),
                                 preferred_element_type=jnp.float32)


def _tc_combine_kernel(s_ref, meta_ref, out_ref):
    s = s_ref[...]
    meta = meta_ref[...]
    match = lax.slice(meta, (0, 0), (_V, _J))
    cnt = lax.slice(meta, (_V, 0), (_V + 1, _J))
    num = jnp.sum(s * match, axis=(0, 1), keepdims=True)
    colsum = jnp.sum(s, axis=0, keepdims=True)
    total = jnp.sum(colsum * cnt, axis=(0, 1), keepdims=True)
    den = total - num
    out_ref[...] = -jnp.log(num / (den + _EPS) + _EPS2)


@jax.jit
def kernel(x, track_idxs, y):
    ti = track_idxs.astype(jnp.int32)
    y2 = y.reshape(_J, _D)
    sc = pl.kernel(
        _sc_meta_kernel,
        out_type=jax.ShapeDtypeStruct((_MR, _J), jnp.float32),
        mesh=plsc.VectorSubcoreMesh(core_axis_name="c", subcore_axis_name="s",
                                    num_cores=_NC, num_subcores=_NS),
        compiler_params=pltpu.CompilerParams(needs_layout_passes=False),
        scratch_types=[
            pltpu.VMEM((_N,), jnp.int32),
            pltpu.VMEM((_V,), jnp.int32),
            pltpu.VMEM((_V,), jnp.int32),
            pltpu.VMEM((_V,), jnp.int32),
            pltpu.VMEM((_MR, _J), jnp.float32),
        ],
    )
    meta = sc(ti)
    s = pl.pallas_call(
        _tc_dense_kernel,
        out_shape=jax.ShapeDtypeStruct((_V, _J), jnp.float32),
    )(x, y, ti.astype(jnp.int8).reshape(_N, 1))
    out = pl.pallas_call(
        _tc_combine_kernel,
        out_shape=jax.ShapeDtypeStruct((1, 1), jnp.float32),
    )(s, meta)
    return out.reshape(1)


# speedup vs baseline: 1.1656x; 1.0346x over previous
import jax
import jax.numpy as jnp
from jax import lax
from jax.experimental import pallas as pl
from jax.experimental.pallas import tpu as pltpu
from jax.experimental.pallas import tpu_sc as plsc

_TEMP = 0.3
_EPS = 1e-09
_EPS2 = 1e-10
_N = 4096
_V = 64
_Q = 2
_D = 64
_J = _V * _Q
_NC = 2
_NS = 16
_MR = 72


def _sc_meta_kernel(ti_hbm, meta_hbm,
                    ti_v, flags_v, rank_v, utp_v, meta_v):
    cid = lax.axis_index("c")
    sid = lax.axis_index("s")
    wid = sid * _NC + cid
    zeros16 = jnp.zeros((16,), jnp.int32)
    ones16 = jnp.ones((16,), jnp.int32)

    @pl.when(wid == 0)
    def _():
        pltpu.sync_copy(ti_hbm, ti_v)
        zf = jnp.zeros((16,), jnp.float32)
        onesf = jnp.ones((16,), jnp.float32)

        def _zero_row(r, _):
            for c in range(_J // 16):
                meta_v[r, pl.ds(c * 16, 16)] = zf
            return 0

        lax.fori_loop(0, _V + 1, _zero_row, 0, unroll=False)
        for c in range(_V // 16):
            flags_v[pl.ds(c * 16, 16)] = zeros16
            utp_v[pl.ds(c * 16, 16)] = zeros16

        def _presence(c, _):
            t = ti_v[pl.ds(c * 16, 16)]
            plsc.store_scatter(flags_v, [t], ones16)
            return 0

        lax.fori_loop(0, _N // 16, _presence, 0, unroll=False)
        carry = jnp.int32(0)
        for c in range(_V // 16):
            ch = flags_v[pl.ds(c * 16, 16)]
            inc = plsc.cumsum(ch)
            rank_v[pl.ds(c * 16, 16)] = inc - ch + carry
            carry = carry + jnp.sum(ch)
        u_cnt = carry
        for c in range(_V // 16):
            ch = flags_v[pl.ds(c * 16, 16)]
            rk = rank_v[pl.ds(c * 16, 16)]
            vals = lax.iota(jnp.int32, 16) + (c * 16)
            plsc.store_scatter(utp_v, [rk], vals, mask=(ch != 0))
        for c in range(_V // 16):
            vrow = lax.iota(jnp.int32, 16) + (c * 16)
            r = rank_v[pl.ds(c * 16, 16)]
            u1 = plsc.load_gather(utp_v, [lax.shift_right_logical(r, 1)])
            k1 = u1 * 2 + (r & 1)
            r2 = r + u_cnt
            u2 = plsc.load_gather(utp_v, [lax.shift_right_logical(r2, 1)])
            k2 = u2 * 2 + (r2 & 1)
            plsc.store_scatter(meta_v, [vrow, k1], onesf)
            plsc.store_scatter(meta_v, [vrow, k2], onesf)
        row64 = jnp.full((16,), _V, dtype=jnp.int32)
        for c in range(_J // 16):
            j = lax.iota(jnp.int32, 16) + c * 16
            s = lax.shift_right_logical(j, 1)
            u = plsc.load_gather(utp_v, [s])
            g = u * 2 + (j & 1)
            validm = j < 2 * u_cnt
            plsc.addupdate_scatter(meta_v, [row64, g], onesf, mask=validm)
        pltpu.sync_copy(meta_v, meta_hbm)


def _tc_dense_kernel(x_ref, y_ref, ti_ref, s_ref):
    x = x_ref[...]
    y2 = y_ref[...].reshape(_J, _D)
    ti = ti_ref[...].astype(jnp.int32)
    logits = lax.dot_general(x, y2, (((1,), (1,)), ((), ())),
                             preferred_element_type=jnp.float32)
    e = jnp.exp(logits * (1.0 / _TEMP))
    vals = lax.broadcasted_iota(jnp.int32, (1, _V), 1)
    onehot = (ti == vals).astype(jnp.float32)
    s_ref[...] = lax.dot_general(onehot, e, (((0,), (0,)), ((), ())),
                                 preferred_element_type=jnp.float32)


def _tc_combine_kernel(s_ref, meta_ref, out_ref):
    s = s_ref[...]
    meta = meta_ref[...]
    match = lax.slice(meta, (0, 0), (_V, _J))
    cnt = lax.slice(meta, (_V, 0), (_V + 1, _J))
    num = jnp.sum(s * match, axis=(0, 1), keepdims=True)
    colsum = jnp.sum(s, axis=0, keepdims=True)
    total = jnp.sum(colsum * cnt, axis=(0, 1), keepdims=True)
    den = total - num
    out_ref[...] = -jnp.log(num / (den + _EPS) + _EPS2)


@jax.jit
def kernel(x, track_idxs, y):
    ti = track_idxs.astype(jnp.int32)
    y2 = y.reshape(_J, _D)
    sc = pl.kernel(
        _sc_meta_kernel,
        out_type=jax.ShapeDtypeStruct((_MR, _J), jnp.float32),
        mesh=plsc.VectorSubcoreMesh(core_axis_name="c", subcore_axis_name="s",
                                    num_cores=_NC, num_subcores=_NS),
        compiler_params=pltpu.CompilerParams(needs_layout_passes=False),
        scratch_types=[
            pltpu.VMEM((_N,), jnp.int32),
            pltpu.VMEM((_V,), jnp.int32),
            pltpu.VMEM((_V,), jnp.int32),
            pltpu.VMEM((_V,), jnp.int32),
            pltpu.VMEM((_MR, _J), jnp.float32),
        ],
    )
    meta = sc(ti)
    s = pl.pallas_call(
        _tc_dense_kernel,
        out_shape=jax.ShapeDtypeStruct((_V, _J), jnp.float32),
    )(x, y, ti.astype(jnp.int8).reshape(_N, 1))
    out = pl.pallas_call(
        _tc_combine_kernel,
        out_shape=jax.ShapeDtypeStruct((1, 1), jnp.float32),
    )(s, meta)
    return out.reshape(1)
